# baseline (speedup 1.0000x reference)
import dataclasses
import functools

import jax
import jax.numpy as jnp
from jax import lax
from jax.experimental import pallas as pl
from jax.experimental.pallas import tpu as pltpu
from jax.experimental.pallas import tpu_sc as plsc

N = 10000
N_PAD = 10240
F = 128
NW = 32
NT = 16
CHUNK = 112
CPW = 96
E_PAD = NW * CPW * CHUNK
ROWS_PT = N_PAD // NT
BLK = 512
GRID = N_PAD // BLK


def _mm_body(x_ref, w_ref, a_ref, h_ref, aa_ref):
    h = jnp.dot(x_ref[...], w_ref[...], preferred_element_type=jnp.float32)
    h_ref[...] = h.astype(jnp.bfloat16)
    aa_ref[...] = jnp.dot(h, a_ref[...], preferred_element_type=jnp.float32)


def _layer_matmul(x, W, A):
    return pl.pallas_call(
        _mm_body,
        grid=(GRID,),
        in_specs=[
            pl.BlockSpec((BLK, F), lambda i: (i, 0)),
            pl.BlockSpec((F, F), lambda i: (0, 0)),
            pl.BlockSpec((F, F), lambda i: (0, 0)),
        ],
        out_specs=[
            pl.BlockSpec((BLK, F), lambda i: (i, 0)),
            pl.BlockSpec((BLK, F), lambda i: (i, 0)),
        ],
        out_shape=[
            jax.ShapeDtypeStruct((N_PAD, F), jnp.bfloat16),
            jax.ShapeDtypeStruct((N_PAD, F), jnp.float32),
        ],
    )(x, W, A)


def _x2_block(a0, a1, s_ref, b, m):
    s = jnp.sum(s_ref[...], axis=0)[:, None]
    return jnp.maximum((a0[...] + a1[...]) / (s + 1e-16) + b[...], 0.0) * m[...]


def _fin_mm_body(a0, a1, s_ref, b, m, w_ref, aa_ref, h_ref, aa_out):
    x2 = _x2_block(a0, a1, s_ref, b, m)
    h = jnp.dot(x2, w_ref[...], preferred_element_type=jnp.float32)
    h_ref[...] = h.astype(jnp.bfloat16)
    aa_out[...] = jnp.dot(h, aa_ref[...], preferred_element_type=jnp.float32)


def _finalize_matmul(acc0, acc1, s_parts, b, mask, W, A):
    return pl.pallas_call(
        _fin_mm_body,
        grid=(GRID,),
        in_specs=[
            pl.BlockSpec((BLK, F), lambda i: (i, 0)),
            pl.BlockSpec((BLK, F), lambda i: (i, 0)),
            pl.BlockSpec((2, BLK), lambda i: (0, i)),
            pl.BlockSpec((1, F), lambda i: (0, 0)),
            pl.BlockSpec((BLK, 1), lambda i: (i, 0)),
            pl.BlockSpec((F, F), lambda i: (0, 0)),
            pl.BlockSpec((F, F), lambda i: (0, 0)),
        ],
        out_specs=[
            pl.BlockSpec((BLK, F), lambda i: (i, 0)),
            pl.BlockSpec((BLK, F), lambda i: (i, 0)),
        ],
        out_shape=[
            jax.ShapeDtypeStruct((N_PAD, F), jnp.bfloat16),
            jax.ShapeDtypeStruct((N_PAD, F), jnp.float32),
        ],
    )(acc0, acc1, s_parts, b, mask, W, A)


def _fin_mean_body(a0, a1, s_ref, b, m, o_ref):
    i = pl.program_id(0)
    x2 = _x2_block(a0, a1, s_ref, b, m)

    @pl.when(i == 0)
    def _():
        o_ref[...] = jnp.zeros_like(o_ref)

    o_ref[...] += jnp.sum(x2, axis=0, keepdims=True)

    @pl.when(i == GRID - 1)
    def _():
        o_ref[...] = o_ref[...] * (1.0 / N)


def _finalize_mean(acc0, acc1, s_parts, b, mask):
    return pl.pallas_call(
        _fin_mean_body,
        grid=(GRID,),
        in_specs=[
            pl.BlockSpec((BLK, F), lambda i: (i, 0)),
            pl.BlockSpec((BLK, F), lambda i: (i, 0)),
            pl.BlockSpec((2, BLK), lambda i: (0, i)),
            pl.BlockSpec((1, F), lambda i: (0, 0)),
            pl.BlockSpec((BLK, 1), lambda i: (i, 0)),
        ],
        out_specs=pl.BlockSpec((1, F), lambda i: (0, 0)),
        out_shape=jax.ShapeDtypeStruct((1, F), jnp.float32),
    )(acc0, acc1, s_parts, b, mask)


_SC_MESH = plsc.VectorSubcoreMesh(core_axis_name="c", subcore_axis_name="s")

_SC_PARAMS = pltpu.CompilerParams(use_tc_tiling_on_sc=False)
if "needs_layout_passes" in pltpu.CompilerParams.__dataclass_fields__:
    _SC_PARAMS = dataclasses.replace(_SC_PARAMS, needs_layout_passes=False)


@functools.partial(
    pl.kernel,
    compiler_params=_SC_PARAMS,
    out_type=[
        jax.ShapeDtypeStruct((2, N_PAD, F), jnp.float32),
        jax.ShapeDtypeStruct((2, N_PAD), jnp.float32),
    ],
    mesh=_SC_MESH,
    scratch_types=[
        [pltpu.VMEM((CHUNK,), jnp.int32) for _ in range(4)],
        [pltpu.VMEM((CHUNK,), jnp.int32) for _ in range(4)],
        [pltpu.VMEM((CHUNK,), jnp.float32) for _ in range(2)],
        [pltpu.VMEM((CHUNK,), jnp.float32) for _ in range(2)],
        [pltpu.VMEM((CHUNK,), jnp.float32) for _ in range(2)],
        [pltpu.VMEM((CHUNK, F // 2), jnp.uint32) for _ in range(2)],
        [pltpu.VMEM((CHUNK, F), jnp.float32) for _ in range(2)],
        pltpu.VMEM((ROWS_PT,), jnp.float32),
        [pltpu.SemaphoreType.DMA for _ in range(4)],
        [pltpu.SemaphoreType.DMA for _ in range(2)],
        [pltpu.SemaphoreType.DMA for _ in range(2)],
        pltpu.VMEM_SHARED((N_PAD, F), jnp.float32),
        pltpu.VMEM_SHARED((N_PAD,), jnp.float32),
    ],
)
def _sc_edge_kernel(h_hbm, as_hbm, ad_hbm, src_hbm, dst_hbm,
                    acc_hbm, s_hbm,
                    src_r, dst_r, asg_r, adg_r, w_r, rb16, rf32, zs,
                    sem_i, sem_g, sem_s, acc_sh, s_sh):
    c = lax.axis_index("c")
    t = lax.axis_index("s")
    wid = c * NT + t

    def issue_idx(j, p):
        pltpu.async_copy(src_hbm.at[wid, j], src_r[p], sem_i[p])
        pltpu.async_copy(dst_hbm.at[wid, j], dst_r[p], sem_i[p])

    def wait_idx(j, p):
        pltpu.make_async_copy(src_hbm.at[wid, j], src_r[p], sem_i[p]).wait()
        pltpu.make_async_copy(dst_hbm.at[wid, j], dst_r[p], sem_i[p]).wait()

    def issue_gathers(b, p):
        pltpu.async_copy(h_hbm.at[src_r[p]], rb16[b], sem_g[b])
        pltpu.async_copy(as_hbm.at[src_r[p]], asg_r[b], sem_g[b])
        pltpu.async_copy(ad_hbm.at[dst_r[p]], adg_r[b], sem_g[b])

    def wait_gathers(b, p):
        pltpu.make_async_copy(h_hbm.at[src_r[p]], rb16[b], sem_g[b]).wait()
        pltpu.make_async_copy(as_hbm.at[src_r[p]], asg_r[b], sem_g[b]).wait()
        pltpu.make_async_copy(ad_hbm.at[dst_r[p]], adg_r[b], sem_g[b]).wait()

    def issue_scatters(b, p):
        pltpu.async_copy(rf32[b], acc_sh.at[dst_r[p]], sem_s[b], add=True)
        pltpu.async_copy(w_r[b], s_sh.at[dst_r[p]], sem_s[b], add=True)

    def wait_scatters(b, p):
        pltpu.make_async_copy(rf32[b], acc_sh.at[dst_r[p]], sem_s[b]).wait()
        pltpu.make_async_copy(w_r[b], s_sh.at[dst_r[p]], sem_s[b]).wait()

    @pl.loop(0, CHUNK)
    def _(r):
        for k in range(F // 16):
            rf32[0][r, pl.ds(k * 16, 16)] = jnp.zeros((16,), jnp.float32)

    @pl.loop(0, ROWS_PT // 64)
    def _(i):
        pltpu.sync_copy(rf32[0].at[pl.ds(0, 64)],
                        acc_sh.at[pl.ds(t * ROWS_PT + i * 64, 64)])

    @pl.loop(0, ROWS_PT // 16)
    def _(i):
        zs[pl.ds(i * 16, 16)] = jnp.zeros((16,), jnp.float32)

    pltpu.sync_copy(zs, s_sh.at[pl.ds(t * ROWS_PT, ROWS_PT)])

    plsc.subcore_barrier()

    pltpu.sync_copy(src_hbm.at[wid, 0], src_r[0])
    pltpu.sync_copy(dst_hbm.at[wid, 0], dst_r[0])
    pltpu.sync_copy(src_hbm.at[wid, 1], src_r[1])
    pltpu.sync_copy(dst_hbm.at[wid, 1], dst_r[1])
    issue_gathers(0, 0)

    hi_mask = jnp.uint32(0xFFFF0000)

    @pl.loop(0, CPW, step=4)
    def _(j0):
        for b4 in range(4):
            j = j0 + b4
            b2 = b4 % 2
            wait_gathers(b2, b4)

            @pl.when(j >= 2)
            def _():
                wait_scatters(b2, (b4 + 2) % 4)

            @pl.loop(0, CHUNK // 16)
            def _(k):
                sl = pl.ds(k * 16, 16)
                e = asg_r[b2][sl] + adg_r[b2][sl]
                e = jnp.maximum(e, 0.2 * e)
                w = jnp.exp(e)
                w_r[b2][sl] = w
                base = k * 16
                for lane in range(16):
                    ws = w[lane]
                    r = base + lane
                    for wd in range(F // 32):
                        u = rb16[b2][r, pl.ds(wd * 16, 16)]
                        lo = plsc.bitcast(u << 16, jnp.float32)
                        hi = plsc.bitcast(u & hi_mask, jnp.float32)
                        rf32[b2][r, pl.ds(wd * 32, 16)] = lo * ws
                        rf32[b2][r, pl.ds(wd * 32 + 16, 16)] = hi * ws

            issue_scatters(b2, b4)

            @pl.when(j + 2 < CPW)
            def _():
                issue_idx(j + 2, (b4 + 2) % 4)

            @pl.when(j + 1 < CPW)
            def _():
                @pl.when(j >= 1)
                def _():
                    wait_idx(j + 1, (b4 + 1) % 4)

                issue_gathers(1 - b2, (b4 + 1) % 4)

    wait_scatters(0, (CPW - 2) % 4)
    wait_scatters(1, (CPW - 1) % 4)

    plsc.subcore_barrier()

    pltpu.sync_copy(acc_sh.at[pl.ds(t * ROWS_PT, ROWS_PT)],
                    acc_hbm.at[c, pl.ds(t * ROWS_PT, ROWS_PT)])
    pltpu.sync_copy(s_sh.at[pl.ds(t * ROWS_PT, ROWS_PT)],
                    s_hbm.at[c, pl.ds(t * ROWS_PT, ROWS_PT)])


def kernel(x, edge_index, W1, a_src1, a_dst1, b1, W2, a_src2, a_dst2, b2):
    loops = jnp.arange(N, dtype=edge_index.dtype)
    src = jnp.concatenate([edge_index[0], loops]).astype(jnp.int32)
    dst = jnp.concatenate([edge_index[1], loops]).astype(jnp.int32)
    pad = E_PAD - src.shape[0]
    pad_idx = N + (jnp.arange(pad, dtype=jnp.int32) % (N_PAD - N))
    src_t = jnp.concatenate([src, pad_idx]).reshape(NW, CPW, CHUNK)
    dst_t = jnp.concatenate([dst, pad_idx]).reshape(NW, CPW, CHUNK)

    x_pad = jnp.zeros((N_PAD, F), jnp.float32).at[:N].set(x)
    mask = (jnp.arange(N_PAD) < N).astype(jnp.float32).reshape(N_PAD, 1)

    def colmat(a_s, a_d):
        m = jnp.zeros((F, F), jnp.float32)
        return m.at[:, 0].set(a_s).at[:, 1].set(a_d)

    r16 = jnp.arange(16, dtype=jnp.int32)
    q = jnp.concatenate(
        [jnp.stack([32 * wd + r16, 32 * wd + 16 + r16], axis=1).reshape(32)
         for wd in range(F // 32)])

    A1 = colmat(a_src1, a_dst1)[q]
    A2 = colmat(a_src2, a_dst2)[q]
    W1q = W1[:, q]
    W2q = W2[:, q]
    b1r = b1.reshape(1, F)
    b2r = b2.reshape(1, F)

    def pack_u32(h):
        return jax.lax.bitcast_convert_type(
            h.reshape(N_PAD, F // 2, 2), jnp.uint32)

    h1, aa1 = _layer_matmul(x_pad, W1q, A1)
    acc1, s1 = _sc_edge_kernel(pack_u32(h1), aa1[:, 0], aa1[:, 1], src_t, dst_t)

    h2, aa2 = _finalize_matmul(acc1[0], acc1[1], s1, b1r, mask, W2q, A2)
    acc2, s2 = _sc_edge_kernel(pack_u32(h2), aa2[:, 0], aa2[:, 1], src_t, dst_t)

    return _finalize_mean(acc2[0], acc2[1], s2, b2r, mask)


# speedup vs baseline: 1.9866x; 1.9866x over previous
import dataclasses
import functools

import jax
import jax.numpy as jnp
from jax import lax
from jax.experimental import pallas as pl
from jax.experimental.pallas import tpu as pltpu
from jax.experimental.pallas import tpu_sc as plsc

N = 10000
N_PAD = 10240
F = 128
NW = 32
NT = 16
CHUNK = 112
CPW = 93
E_PAD = NW * CPW * CHUNK
ROWS_PT = N_PAD // NT
BLK = 512
GRID = N_PAD // BLK


def _mm_body(x_ref, w_ref, a_ref, h_ref, aa_ref):
    h = jnp.dot(x_ref[...], w_ref[...], preferred_element_type=jnp.float32)
    h_ref[...] = h
    aa_ref[...] = jnp.dot(h, a_ref[...], preferred_element_type=jnp.float32)


def _layer_matmul(x, W, A):
    return pl.pallas_call(
        _mm_body,
        grid=(GRID,),
        in_specs=[
            pl.BlockSpec((BLK, F), lambda i: (i, 0)),
            pl.BlockSpec((F, F), lambda i: (0, 0)),
            pl.BlockSpec((F, F), lambda i: (0, 0)),
        ],
        out_specs=[
            pl.BlockSpec((BLK, F), lambda i: (i, 0)),
            pl.BlockSpec((BLK, F), lambda i: (i, 0)),
        ],
        out_shape=[
            jax.ShapeDtypeStruct((N_PAD, F), jnp.float32),
            jax.ShapeDtypeStruct((N_PAD, F), jnp.float32),
        ],
    )(x, W, A)


def _x2_block(a0, a1, s_ref, b, m):
    s = jnp.sum(s_ref[...], axis=0)[:, None]
    return jnp.maximum((a0[...] + a1[...]) / (s + 1e-16) + b[...], 0.0) * m[...]


def _fin_mm_body(a0, a1, s_ref, b, m, w_ref, aa_ref, h_ref, aa_out):
    x2 = _x2_block(a0, a1, s_ref, b, m)
    h = jnp.dot(x2, w_ref[...], preferred_element_type=jnp.float32)
    h_ref[...] = h
    aa_out[...] = jnp.dot(h, aa_ref[...], preferred_element_type=jnp.float32)


def _finalize_matmul(acc0, acc1, s_parts, b, mask, W, A):
    return pl.pallas_call(
        _fin_mm_body,
        grid=(GRID,),
        in_specs=[
            pl.BlockSpec((BLK, F), lambda i: (i, 0)),
            pl.BlockSpec((BLK, F), lambda i: (i, 0)),
            pl.BlockSpec((2, BLK), lambda i: (0, i)),
            pl.BlockSpec((1, F), lambda i: (0, 0)),
            pl.BlockSpec((BLK, 1), lambda i: (i, 0)),
            pl.BlockSpec((F, F), lambda i: (0, 0)),
            pl.BlockSpec((F, F), lambda i: (0, 0)),
        ],
        out_specs=[
            pl.BlockSpec((BLK, F), lambda i: (i, 0)),
            pl.BlockSpec((BLK, F), lambda i: (i, 0)),
        ],
        out_shape=[
            jax.ShapeDtypeStruct((N_PAD, F), jnp.float32),
            jax.ShapeDtypeStruct((N_PAD, F), jnp.float32),
        ],
    )(acc0, acc1, s_parts, b, mask, W, A)


def _fin_mean_body(a0, a1, s_ref, b, m, o_ref):
    i = pl.program_id(0)
    x2 = _x2_block(a0, a1, s_ref, b, m)

    @pl.when(i == 0)
    def _():
        o_ref[...] = jnp.zeros_like(o_ref)

    o_ref[...] += jnp.sum(x2, axis=0, keepdims=True)

    @pl.when(i == GRID - 1)
    def _():
        o_ref[...] = o_ref[...] * (1.0 / N)


def _finalize_mean(acc0, acc1, s_parts, b, mask):
    return pl.pallas_call(
        _fin_mean_body,
        grid=(GRID,),
        in_specs=[
            pl.BlockSpec((BLK, F), lambda i: (i, 0)),
            pl.BlockSpec((BLK, F), lambda i: (i, 0)),
            pl.BlockSpec((2, BLK), lambda i: (0, i)),
            pl.BlockSpec((1, F), lambda i: (0, 0)),
            pl.BlockSpec((BLK, 1), lambda i: (i, 0)),
        ],
        out_specs=pl.BlockSpec((1, F), lambda i: (0, 0)),
        out_shape=jax.ShapeDtypeStruct((1, F), jnp.float32),
    )(acc0, acc1, s_parts, b, mask)


_SC_MESH = plsc.VectorSubcoreMesh(core_axis_name="c", subcore_axis_name="s")

_SC_PARAMS = pltpu.CompilerParams()
if "needs_layout_passes" in pltpu.CompilerParams.__dataclass_fields__:
    _SC_PARAMS = dataclasses.replace(_SC_PARAMS, needs_layout_passes=False)


@functools.partial(
    pl.kernel,
    compiler_params=_SC_PARAMS,
    out_type=[
        jax.ShapeDtypeStruct((2, N_PAD, F), jnp.float32),
        jax.ShapeDtypeStruct((2, N_PAD), jnp.float32),
    ],
    mesh=_SC_MESH,
    scratch_types=[
        [pltpu.VMEM((CHUNK,), jnp.int32) for _ in range(3)],
        [pltpu.VMEM((CHUNK,), jnp.int32) for _ in range(3)],
        [pltpu.VMEM((CHUNK,), jnp.float32) for _ in range(3)],
        [pltpu.VMEM((CHUNK,), jnp.float32) for _ in range(3)],
        [pltpu.VMEM((CHUNK,), jnp.float32) for _ in range(3)],
        [pltpu.VMEM((CHUNK, F), jnp.float32) for _ in range(3)],
        pltpu.VMEM((ROWS_PT,), jnp.float32),
        [pltpu.SemaphoreType.DMA for _ in range(3)],
        [pltpu.SemaphoreType.DMA for _ in range(3)],
        [pltpu.SemaphoreType.DMA for _ in range(3)],
        pltpu.VMEM_SHARED((N_PAD, F), jnp.float32),
        pltpu.VMEM_SHARED((N_PAD,), jnp.float32),
    ],
)
def _sc_edge_kernel(h_hbm, as_hbm, ad_hbm, src_hbm, dst_hbm,
                    acc_hbm, s_hbm,
                    src_r, dst_r, asg_r, adg_r, w_r, rows_r, zs,
                    sem_i, sem_g, sem_s, acc_sh, s_sh):
    c = lax.axis_index("c")
    t = lax.axis_index("s")
    wid = c * NT + t

    def issue_idx(j, p):
        pltpu.async_copy(src_hbm.at[wid, j], src_r[p], sem_i[p])
        pltpu.async_copy(dst_hbm.at[wid, j], dst_r[p], sem_i[p])

    def wait_idx(j, p):
        pltpu.make_async_copy(src_hbm.at[wid, j], src_r[p], sem_i[p]).wait()
        pltpu.make_async_copy(dst_hbm.at[wid, j], dst_r[p], sem_i[p]).wait()

    def issue_gathers(p):
        pltpu.async_copy(h_hbm.at[src_r[p]], rows_r[p], sem_g[p])
        pltpu.async_copy(as_hbm.at[src_r[p]], asg_r[p], sem_g[p])
        pltpu.async_copy(ad_hbm.at[dst_r[p]], adg_r[p], sem_g[p])

    def wait_gathers(p):
        pltpu.make_async_copy(h_hbm.at[src_r[p]], rows_r[p], sem_g[p]).wait()
        pltpu.make_async_copy(as_hbm.at[src_r[p]], asg_r[p], sem_g[p]).wait()
        pltpu.make_async_copy(ad_hbm.at[dst_r[p]], adg_r[p], sem_g[p]).wait()

    def issue_scatters(p):
        pltpu.async_copy(rows_r[p], acc_sh.at[dst_r[p]], sem_s[p], add=True)
        pltpu.async_copy(w_r[p], s_sh.at[dst_r[p]], sem_s[p], add=True)

    def wait_scatters(p):
        pltpu.make_async_copy(rows_r[p], acc_sh.at[dst_r[p]], sem_s[p]).wait()
        pltpu.make_async_copy(w_r[p], s_sh.at[dst_r[p]], sem_s[p]).wait()

    @pl.loop(0, CHUNK)
    def _(r):
        for k in range(F // 16):
            rows_r[0][r, pl.ds(k * 16, 16)] = jnp.zeros((16,), jnp.float32)

    @pl.loop(0, ROWS_PT // 64)
    def _(i):
        pltpu.sync_copy(rows_r[0].at[pl.ds(0, 64)],
                        acc_sh.at[pl.ds(t * ROWS_PT + i * 64, 64)])

    @pl.loop(0, ROWS_PT // 16)
    def _(i):
        zs[pl.ds(i * 16, 16)] = jnp.zeros((16,), jnp.float32)

    pltpu.sync_copy(zs, s_sh.at[pl.ds(t * ROWS_PT, ROWS_PT)])

    plsc.subcore_barrier()

    for b in range(2):
        pltpu.sync_copy(src_hbm.at[wid, b], src_r[b])
        pltpu.sync_copy(dst_hbm.at[wid, b], dst_r[b])
        issue_gathers(b)

    @pl.loop(0, CPW, step=3)
    def _(j0):
        for b in range(3):
            j = j0 + b
            bn = (b + 2) % 3
            bp = (b + 1) % 3
            wait_gathers(b)

            @pl.when((j >= 1) & (j + 1 < CPW))
            def _():
                wait_idx(j + 1, bp)
                issue_gathers(bp)

            rows_b = rows_r[b]

            @pl.loop(0, CHUNK // 16)
            def _(k):
                sl = pl.ds(k * 16, 16)
                e = asg_r[b][sl] + adg_r[b][sl]
                e = jnp.maximum(e, 0.2 * e)
                w = jnp.exp(e)
                w_r[b][sl] = w
                base = k * 16
                for lane in range(16):
                    ws = w[lane]
                    r = base + lane
                    for f in range(F // 16):
                        fl = pl.ds(f * 16, 16)
                        rows_b[r, fl] = rows_b[r, fl] * ws

            issue_scatters(b)

            @pl.when(j >= 1)
            def _():
                wait_scatters(bn)

            @pl.when(j + 2 < CPW)
            def _():
                issue_idx(j + 2, bn)

    wait_scatters((CPW - 1) % 3)

    plsc.subcore_barrier()

    pltpu.sync_copy(acc_sh.at[pl.ds(t * ROWS_PT, ROWS_PT)],
                    acc_hbm.at[c, pl.ds(t * ROWS_PT, ROWS_PT)])
    pltpu.sync_copy(s_sh.at[pl.ds(t * ROWS_PT, ROWS_PT)],
                    s_hbm.at[c, pl.ds(t * ROWS_PT, ROWS_PT)])


def kernel(x, edge_index, W1, a_src1, a_dst1, b1, W2, a_src2, a_dst2, b2):
    loops = jnp.arange(N, dtype=edge_index.dtype)
    src = jnp.concatenate([edge_index[0], loops]).astype(jnp.int32)
    dst = jnp.concatenate([edge_index[1], loops]).astype(jnp.int32)
    pad = E_PAD - src.shape[0]
    pad_idx = N + (jnp.arange(pad, dtype=jnp.int32) % (N_PAD - N))
    src_t = jnp.concatenate([src, pad_idx]).reshape(NW, CPW, CHUNK)
    dst_t = jnp.concatenate([dst, pad_idx]).reshape(NW, CPW, CHUNK)

    x_pad = jnp.zeros((N_PAD, F), jnp.float32).at[:N].set(x)
    mask = (jnp.arange(N_PAD) < N).astype(jnp.float32).reshape(N_PAD, 1)

    def colmat(a_s, a_d):
        m = jnp.zeros((F, F), jnp.float32)
        return m.at[:, 0].set(a_s).at[:, 1].set(a_d)

    A1 = colmat(a_src1, a_dst1)
    A2 = colmat(a_src2, a_dst2)
    b1r = b1.reshape(1, F)
    b2r = b2.reshape(1, F)

    h1, aa1 = _layer_matmul(x_pad, W1, A1)
    acc1, s1 = _sc_edge_kernel(h1, aa1[:, 0], aa1[:, 1], src_t, dst_t)

    h2, aa2 = _finalize_matmul(acc1[0], acc1[1], s1, b1r, mask, W2, A2)
    acc2, s2 = _sc_edge_kernel(h2, aa2[:, 0], aa2[:, 1], src_t, dst_t)

    return _finalize_mean(acc2[0], acc2[1], s2, b2r, mask)


# speedup vs baseline: 2.0539x; 1.0338x over previous
import dataclasses
import functools

import jax
import jax.numpy as jnp
from jax import lax
from jax.experimental import pallas as pl
from jax.experimental.pallas import tpu as pltpu
from jax.experimental.pallas import tpu_sc as plsc

N = 10000
N_PAD = 10240
F = 128
NW = 32
NT = 16
CHUNK = 112
CPW = 93
E_PAD = NW * CPW * CHUNK
ROWS_PT = N_PAD // NT
BLK = 512
GRID = N_PAD // BLK


def _mm_body(x_ref, w_ref, a_ref, h_ref, aa_ref):
    h = jnp.dot(x_ref[...], w_ref[...], preferred_element_type=jnp.float32)
    h_ref[...] = h
    aa_ref[...] = jnp.dot(h, a_ref[...], preferred_element_type=jnp.float32)


def _layer_matmul(x, W, A):
    return pl.pallas_call(
        _mm_body,
        grid=(GRID,),
        in_specs=[
            pl.BlockSpec((BLK, F), lambda i: (i, 0)),
            pl.BlockSpec((F, F), lambda i: (0, 0)),
            pl.BlockSpec((F, F), lambda i: (0, 0)),
        ],
        out_specs=[
            pl.BlockSpec((BLK, F), lambda i: (i, 0)),
            pl.BlockSpec((BLK, F), lambda i: (i, 0)),
        ],
        out_shape=[
            jax.ShapeDtypeStruct((N_PAD, F), jnp.float32),
            jax.ShapeDtypeStruct((N_PAD, F), jnp.float32),
        ],
    )(x, W, A)


def _x2_block(a0, a1, s_ref, b, m):
    s = jnp.sum(s_ref[...], axis=0)[:, None]
    return jnp.maximum((a0[...] + a1[...]) / (s + 1e-16) + b[...], 0.0) * m[...]


def _fin_mm_body(a0, a1, s_ref, b, m, w_ref, aa_ref, h_ref, aa_out):
    x2 = _x2_block(a0, a1, s_ref, b, m)
    h = jnp.dot(x2, w_ref[...], preferred_element_type=jnp.float32)
    h_ref[...] = h
    aa_out[...] = jnp.dot(h, aa_ref[...], preferred_element_type=jnp.float32)


def _finalize_matmul(acc0, acc1, s_parts, b, mask, W, A):
    return pl.pallas_call(
        _fin_mm_body,
        grid=(GRID,),
        in_specs=[
            pl.BlockSpec((BLK, F), lambda i: (i, 0)),
            pl.BlockSpec((BLK, F), lambda i: (i, 0)),
            pl.BlockSpec((2, BLK), lambda i: (0, i)),
            pl.BlockSpec((1, F), lambda i: (0, 0)),
            pl.BlockSpec((BLK, 1), lambda i: (i, 0)),
            pl.BlockSpec((F, F), lambda i: (0, 0)),
            pl.BlockSpec((F, F), lambda i: (0, 0)),
        ],
        out_specs=[
            pl.BlockSpec((BLK, F), lambda i: (i, 0)),
            pl.BlockSpec((BLK, F), lambda i: (i, 0)),
        ],
        out_shape=[
            jax.ShapeDtypeStruct((N_PAD, F), jnp.float32),
            jax.ShapeDtypeStruct((N_PAD, F), jnp.float32),
        ],
    )(acc0, acc1, s_parts, b, mask, W, A)


def _fin_mean_body(a0, a1, s_ref, b, m, o_ref):
    i = pl.program_id(0)
    x2 = _x2_block(a0, a1, s_ref, b, m)

    @pl.when(i == 0)
    def _():
        o_ref[...] = jnp.zeros_like(o_ref)

    o_ref[...] += jnp.sum(x2, axis=0, keepdims=True)

    @pl.when(i == GRID - 1)
    def _():
        o_ref[...] = o_ref[...] * (1.0 / N)


def _finalize_mean(acc0, acc1, s_parts, b, mask):
    return pl.pallas_call(
        _fin_mean_body,
        grid=(GRID,),
        in_specs=[
            pl.BlockSpec((BLK, F), lambda i: (i, 0)),
            pl.BlockSpec((BLK, F), lambda i: (i, 0)),
            pl.BlockSpec((2, BLK), lambda i: (0, i)),
            pl.BlockSpec((1, F), lambda i: (0, 0)),
            pl.BlockSpec((BLK, 1), lambda i: (i, 0)),
        ],
        out_specs=pl.BlockSpec((1, F), lambda i: (0, 0)),
        out_shape=jax.ShapeDtypeStruct((1, F), jnp.float32),
    )(acc0, acc1, s_parts, b, mask)


_SC_MESH = plsc.VectorSubcoreMesh(core_axis_name="c", subcore_axis_name="s")

_SC_PARAMS = pltpu.CompilerParams()
if "needs_layout_passes" in pltpu.CompilerParams.__dataclass_fields__:
    _SC_PARAMS = dataclasses.replace(_SC_PARAMS, needs_layout_passes=False)


@functools.partial(
    pl.kernel,
    compiler_params=_SC_PARAMS,
    out_type=[
        jax.ShapeDtypeStruct((N_PAD, F), jnp.float32),
        jax.ShapeDtypeStruct((N_PAD, F), jnp.float32),
        jax.ShapeDtypeStruct((2, N_PAD), jnp.float32),
    ],
    mesh=_SC_MESH,
    scratch_types=[
        [pltpu.VMEM((CHUNK,), jnp.int32) for _ in range(3)],
        [pltpu.VMEM((CHUNK,), jnp.int32) for _ in range(3)],
        [pltpu.VMEM((CHUNK,), jnp.float32) for _ in range(3)],
        [pltpu.VMEM((CHUNK,), jnp.float32) for _ in range(3)],
        [pltpu.VMEM((CHUNK,), jnp.float32) for _ in range(3)],
        [pltpu.VMEM((CHUNK, F), jnp.float32) for _ in range(3)],
        pltpu.VMEM((ROWS_PT,), jnp.float32),
        [pltpu.SemaphoreType.DMA for _ in range(3)],
        [pltpu.SemaphoreType.DMA for _ in range(3)],
        [pltpu.SemaphoreType.DMA for _ in range(3)],
        pltpu.VMEM_SHARED((N_PAD, F), jnp.float32),
        pltpu.VMEM_SHARED((N_PAD,), jnp.float32),
    ],
)
def _sc_edge_kernel(h_hbm, as_hbm, ad_hbm, src_hbm, dst_hbm,
                    acc0_hbm, acc1_hbm, s_hbm,
                    src_r, dst_r, asg_r, adg_r, w_r, rows_r, zs,
                    sem_i, sem_g, sem_s, acc_sh, s_sh):
    c = lax.axis_index("c")
    t = lax.axis_index("s")
    wid = c * NT + t

    def issue_idx(j, p):
        pltpu.async_copy(src_hbm.at[wid, j], src_r[p], sem_i[p])
        pltpu.async_copy(dst_hbm.at[wid, j], dst_r[p], sem_i[p])

    def wait_idx(j, p):
        pltpu.make_async_copy(src_hbm.at[wid, j], src_r[p], sem_i[p]).wait()
        pltpu.make_async_copy(dst_hbm.at[wid, j], dst_r[p], sem_i[p]).wait()

    def issue_gathers(p):
        pltpu.async_copy(h_hbm.at[src_r[p]], rows_r[p], sem_g[p])
        pltpu.async_copy(as_hbm.at[src_r[p]], asg_r[p], sem_g[p])
        pltpu.async_copy(ad_hbm.at[dst_r[p]], adg_r[p], sem_g[p])

    def wait_gathers(p):
        pltpu.make_async_copy(h_hbm.at[src_r[p]], rows_r[p], sem_g[p]).wait()
        pltpu.make_async_copy(as_hbm.at[src_r[p]], asg_r[p], sem_g[p]).wait()
        pltpu.make_async_copy(ad_hbm.at[dst_r[p]], adg_r[p], sem_g[p]).wait()

    def issue_scatters(p):
        pltpu.async_copy(rows_r[p], acc_sh.at[dst_r[p]], sem_s[p], add=True)
        pltpu.async_copy(w_r[p], s_sh.at[dst_r[p]], sem_s[p], add=True)

    def wait_scatters(p):
        pltpu.make_async_copy(rows_r[p], acc_sh.at[dst_r[p]], sem_s[p]).wait()
        pltpu.make_async_copy(w_r[p], s_sh.at[dst_r[p]], sem_s[p]).wait()

    @pl.loop(0, CHUNK)
    def _(r):
        for k in range(F // 16):
            rows_r[0][r, pl.ds(k * 16, 16)] = jnp.zeros((16,), jnp.float32)

    @pl.loop(0, ROWS_PT // 64)
    def _(i):
        pltpu.sync_copy(rows_r[0].at[pl.ds(0, 64)],
                        acc_sh.at[pl.ds(t * ROWS_PT + i * 64, 64)])

    @pl.loop(0, ROWS_PT // 16)
    def _(i):
        zs[pl.ds(i * 16, 16)] = jnp.zeros((16,), jnp.float32)

    pltpu.sync_copy(zs, s_sh.at[pl.ds(t * ROWS_PT, ROWS_PT)])

    plsc.subcore_barrier()

    for b in range(2):
        pltpu.sync_copy(src_hbm.at[wid, b], src_r[b])
        pltpu.sync_copy(dst_hbm.at[wid, b], dst_r[b])
        issue_gathers(b)

    @pl.loop(0, CPW, step=3)
    def _(j0):
        for b in range(3):
            j = j0 + b
            bn = (b + 2) % 3
            bp = (b + 1) % 3
            wait_gathers(b)

            @pl.when((j >= 1) & (j + 1 < CPW))
            def _():
                wait_idx(j + 1, bp)
                issue_gathers(bp)

            rows_b = rows_r[b]

            @pl.loop(0, CHUNK // 16)
            def _(k):
                sl = pl.ds(k * 16, 16)
                e = asg_r[b][sl] + adg_r[b][sl]
                e = jnp.maximum(e, 0.2 * e)
                w = jnp.exp(e)
                w_r[b][sl] = w
                base = k * 16
                for lane in range(16):
                    ws = w[lane]
                    r = base + lane
                    for f in range(F // 16):
                        fl = pl.ds(f * 16, 16)
                        rows_b[r, fl] = rows_b[r, fl] * ws

            issue_scatters(b)

            @pl.when(j >= 1)
            def _():
                wait_scatters(bn)

            @pl.when(j + 2 < CPW)
            def _():
                issue_idx(j + 2, bn)

    wait_scatters((CPW - 1) % 3)

    plsc.subcore_barrier()

    @pl.when(c == 0)
    def _():
        pltpu.sync_copy(acc_sh.at[pl.ds(t * ROWS_PT, ROWS_PT)],
                        acc0_hbm.at[pl.ds(t * ROWS_PT, ROWS_PT)])

    @pl.when(c == 1)
    def _():
        pltpu.sync_copy(acc_sh.at[pl.ds(t * ROWS_PT, ROWS_PT)],
                        acc1_hbm.at[pl.ds(t * ROWS_PT, ROWS_PT)])

    pltpu.sync_copy(s_sh.at[pl.ds(t * ROWS_PT, ROWS_PT)],
                    s_hbm.at[c, pl.ds(t * ROWS_PT, ROWS_PT)])


def kernel(x, edge_index, W1, a_src1, a_dst1, b1, W2, a_src2, a_dst2, b2):
    loops = jnp.arange(N, dtype=edge_index.dtype)
    src = jnp.concatenate([edge_index[0], loops]).astype(jnp.int32)
    dst = jnp.concatenate([edge_index[1], loops]).astype(jnp.int32)
    pad = E_PAD - src.shape[0]
    pad_idx = N + (jnp.arange(pad, dtype=jnp.int32) % (N_PAD - N))
    src_t = jnp.concatenate([src, pad_idx]).reshape(NW, CPW, CHUNK)
    dst_t = jnp.concatenate([dst, pad_idx]).reshape(NW, CPW, CHUNK)

    x_pad = jnp.zeros((N_PAD, F), jnp.float32).at[:N].set(x)
    mask = (jnp.arange(N_PAD) < N).astype(jnp.float32).reshape(N_PAD, 1)

    def colmat(a_s, a_d):
        m = jnp.zeros((F, F), jnp.float32)
        return m.at[:, 0].set(a_s).at[:, 1].set(a_d)

    A1 = colmat(a_src1, a_dst1)
    A2 = colmat(a_src2, a_dst2)
    b1r = b1.reshape(1, F)
    b2r = b2.reshape(1, F)

    h1, aa1 = _layer_matmul(x_pad, W1, A1)
    acc1a, acc1b, s1 = _sc_edge_kernel(h1, aa1[:, 0], aa1[:, 1], src_t, dst_t)

    h2, aa2 = _finalize_matmul(acc1a, acc1b, s1, b1r, mask, W2, A2)
    acc2a, acc2b, s2 = _sc_edge_kernel(h2, aa2[:, 0], aa2[:, 1], src_t, dst_t)

    return _finalize_mean(acc2a, acc2b, s2, b2r, mask)
